# baseline (speedup 1.0000x reference)
import jax
import jax.numpy as jnp
from jax import lax
from jax.experimental import pallas as pl
from jax.experimental.pallas import tpu as pltpu
from jax.experimental.pallas import tpu_sc as plsc

_BATCH = 16384
_DIM = 64
_NC = 2
_NS = 16
_NW = _NC * _NS
_BPW = _BATCH // _NW


def _gather_body(idx_hbm, table_hbm, out_hbm, idx_v, rows_v, outt_v, gsem):
    wid = lax.axis_index("s") * _NC + lax.axis_index("c")
    base = wid * _BPW
    pltpu.sync_copy(idx_hbm.at[pl.ds(base, _BPW)], idx_v)

    @plsc.parallel_loop(0, _BPW // 16, unroll=2)
    def _fire(v):
        vec = idx_v[pl.ds(v * 16, 16)]
        for j in range(16):
            row = vec[j]
            pltpu.async_copy(
                table_hbm.at[pl.ds(row, 1)],
                rows_v.at[pl.ds(v * 16 + j, 1)],
                gsem,
            )

    pltpu.make_async_copy(table_hbm.at[pl.ds(0, _BPW)], rows_v, gsem).wait()

    @pl.loop(0, _DIM)
    def _trans(d):
        dvec = jnp.zeros((16,), jnp.int32) + d
        for kb in range(_BPW // 16):
            kvec = kb * 16 + lax.iota(jnp.int32, 16)
            outt_v[d, pl.ds(kb * 16, 16)] = plsc.load_gather(
                rows_v, [kvec, dvec]
            )

    pltpu.sync_copy(outt_v, out_hbm.at[:, pl.ds(base, _BPW)])


@jax.jit
def _lookup(idx, table):
    mesh = plsc.VectorSubcoreMesh(core_axis_name="c", subcore_axis_name="s")
    return pl.kernel(
        _gather_body,
        out_type=jax.ShapeDtypeStruct((_DIM, _BATCH), jnp.float32),
        mesh=mesh,
        scratch_types=[
            pltpu.VMEM((_BPW,), jnp.int32),
            pltpu.VMEM((_BPW, _DIM), jnp.float32),
            pltpu.VMEM((_DIM, _BPW), jnp.float32),
            pltpu.SemaphoreType.DMA,
        ],
        compiler_params=pltpu.CompilerParams(needs_layout_passes=False),
    )(idx, table)


def kernel(topology_ids, embedding_table):
    return _lookup(topology_ids.astype(jnp.int32), embedding_table).T


# speedup vs baseline: 1.2281x; 1.2281x over previous
import jax
import jax.numpy as jnp
from jax import lax
from jax.experimental import pallas as pl
from jax.experimental.pallas import tpu as pltpu
from jax.experimental.pallas import tpu_sc as plsc

_BATCH = 16384
_DIM = 64
_NC = 2
_NS = 16
_NW = _NC * _NS
_BPW = _BATCH // _NW


_NCH = 4
_CHUNK = _BPW // _NCH


def _gather_body(idx_hbm, table_hbm, out_hbm, idx_v, rows_v, sems, wsem):
    wid = lax.axis_index("s") * _NC + lax.axis_index("c")
    base = wid * _BPW
    pltpu.sync_copy(idx_hbm.at[pl.ds(base, _BPW)], idx_v)

    for c in range(_NCH):

        @plsc.parallel_loop(0, _CHUNK // 16, unroll=2)
        def _fire(v, c=c):
            k = c * _CHUNK + v * 16
            vec = idx_v[pl.ds(k, 16)]
            for j in range(16):
                row = vec[j]
                pltpu.async_copy(
                    table_hbm.at[pl.ds(row, 1)],
                    rows_v.at[pl.ds(k + j, 1)],
                    sems.at[c],
                )

    for c in range(_NCH):
        chunk = rows_v.at[pl.ds(c * _CHUNK, _CHUNK)]
        pltpu.make_async_copy(table_hbm.at[pl.ds(0, _CHUNK)], chunk, sems.at[c]).wait()
        pltpu.async_copy(chunk, out_hbm.at[pl.ds(base + c * _CHUNK, _CHUNK)], wsem)

    pltpu.make_async_copy(table_hbm.at[pl.ds(0, _BPW)], rows_v, wsem).wait()


@jax.jit
def _lookup(idx, table):
    mesh = plsc.VectorSubcoreMesh(core_axis_name="c", subcore_axis_name="s")
    return pl.kernel(
        _gather_body,
        out_type=jax.ShapeDtypeStruct((_BATCH, _DIM), jnp.float32),
        mesh=mesh,
        scratch_types=[
            pltpu.VMEM((_BPW,), jnp.int32),
            pltpu.VMEM((_BPW, _DIM), jnp.float32),
            pltpu.SemaphoreType.DMA((_NCH,)),
            pltpu.SemaphoreType.DMA,
        ],
    )(idx, table)


def kernel(topology_ids, embedding_table):
    return _lookup(topology_ids.astype(jnp.int32), embedding_table)


# speedup vs baseline: 1.2478x; 1.0160x over previous
import jax
import jax.numpy as jnp
from jax import lax
from jax.experimental import pallas as pl
from jax.experimental.pallas import tpu as pltpu
from jax.experimental.pallas import tpu_sc as plsc

_BATCH = 16384
_DIM = 64
_NC = 2
_NS = 16
_NW = _NC * _NS
_BPW = _BATCH // _NW


def _gather_body(idx_hbm, table_hbm, out_hbm, idx_v, rows_v, gsem):
    wid = lax.axis_index("s") * _NC + lax.axis_index("c")
    base = wid * _BPW
    pltpu.sync_copy(idx_hbm.at[pl.ds(base, _BPW)], idx_v)

    @plsc.parallel_loop(0, _BPW // 16, unroll=2)
    def _fire(v):
        vec = idx_v[pl.ds(v * 16, 16)]
        for j in range(16):
            row = vec[j]
            pltpu.async_copy(
                table_hbm.at[pl.ds(row, 1)],
                rows_v.at[pl.ds(v * 16 + j, 1)],
                gsem,
            )

    pltpu.make_async_copy(table_hbm.at[pl.ds(0, _BPW)], rows_v, gsem).wait()
    pltpu.sync_copy(rows_v, out_hbm.at[pl.ds(base, _BPW)])


@jax.jit
def _lookup(idx, table):
    mesh = plsc.VectorSubcoreMesh(core_axis_name="c", subcore_axis_name="s")
    return pl.kernel(
        _gather_body,
        out_type=jax.ShapeDtypeStruct((_BATCH, _DIM), jnp.float32),
        mesh=mesh,
        scratch_types=[
            pltpu.VMEM((_BPW,), jnp.int32),
            pltpu.VMEM((_BPW, _DIM), jnp.float32),
            pltpu.SemaphoreType.DMA,
        ],
    )(idx, table)


def kernel(topology_ids, embedding_table):
    return _lookup(topology_ids.astype(jnp.int32), embedding_table)
